# baseline (speedup 1.0000x reference)
import functools

import jax
import jax.numpy as jnp
from jax import lax
from jax.experimental import pallas as pl
from jax.experimental.pallas import tpu as pltpu
from jax.experimental.pallas import tpu_sc as plsc

N = 10000
E = 320000
D = 128
NUM_LAYERS = 5

NC = 2
NS = 16
NW = NC * NS
CH = 128
CPT = 80
EPT = CPT * CH
E_PAD = NW * EPT
K = 2
SB = 16
NSB = CPT // SB
ROWS_PER_TILE = 640
AGG_ROWS = NS * ROWS_PER_TILE


DO_GATHER = True
DO_SCATTER = True


def _sc_agg_body(h_hbm, src_hbm, dst_hbm, zeros_hbm, out_hbm,
                 agg_sh, src_v, dst_v, rows_v, sem):
    c = lax.axis_index("c")
    s = lax.axis_index("s")
    wid = c * NS + s

    pltpu.sync_copy(zeros_hbm, agg_sh.at[pl.ds(s * ROWS_PER_TILE, ROWS_PER_TILE)])

    pltpu.sync_copy(src_hbm.at[wid], src_v)
    pltpu.sync_copy(dst_hbm.at[wid], dst_v)
    plsc.subcore_barrier()

    @pl.loop(0, CPT)
    def _(ci):
        if DO_GATHER:
            pltpu.async_copy(h_hbm.at[src_v.at[ci]], rows_v, sem).wait()
        if DO_SCATTER:
            pltpu.sync_copy(rows_v, agg_sh.at[dst_v.at[ci]], add=True)

    plsc.subcore_barrier()
    pltpu.sync_copy(agg_sh.at[pl.ds(s * ROWS_PER_TILE, ROWS_PER_TILE)],
                    out_hbm.at[c, pl.ds(s * ROWS_PER_TILE, ROWS_PER_TILE)])


_sc_agg = pl.kernel(
    _sc_agg_body,
    out_type=jax.ShapeDtypeStruct((NC, AGG_ROWS, D), jnp.float32),
    mesh=plsc.VectorSubcoreMesh(core_axis_name="c", subcore_axis_name="s",
                                num_cores=NC, num_subcores=NS),
    scratch_types=[
        pltpu.VMEM_SHARED((AGG_ROWS, D), jnp.float32),
        pltpu.VMEM((CPT, CH), jnp.int32),
        pltpu.VMEM((CPT, CH), jnp.int32),
        pltpu.VMEM((CH, D), jnp.float32),
        pltpu.SemaphoreType.DMA,
    ],
)


def _encoder_body(x_ref, w_ref, b_ref, o_ref):
    o_ref[...] = (
        jnp.dot(x_ref[...], w_ref[...], preferred_element_type=jnp.float32)
        + b_ref[...]
    )


_encoder = pl.pallas_call(
    _encoder_body,
    out_shape=jax.ShapeDtypeStruct((N, D), jnp.float32),
)


def _layer_body(h_ref, a_ref, w1_ref, b1_ref, w2_ref, b2_ref, o_ref, *, last):
    z = h_ref[...] + a_ref[0, :N, :] + a_ref[1, :N, :]
    z = jnp.dot(z, w1_ref[...], preferred_element_type=jnp.float32) + b1_ref[...]
    z = jnp.maximum(z, 0.0)
    z = jnp.dot(z, w2_ref[...], preferred_element_type=jnp.float32) + b2_ref[...]
    if not last:
        z = jnp.maximum(z, 0.0)
    o_ref[...] = z


def _layer_call(last):
    return pl.pallas_call(
        functools.partial(_layer_body, last=last),
        out_shape=jax.ShapeDtypeStruct((N, D), jnp.float32),
    )


_layer_mid = _layer_call(False)
_layer_last = _layer_call(True)


@jax.jit
def _run(x, src_p, dst_p, W_in, b_in, W1, b1, W2, b2):
    zeros = jnp.zeros((ROWS_PER_TILE, D), jnp.float32)
    h = _encoder(x, W_in, b_in.reshape(1, D))
    for l in range(NUM_LAYERS):
        agg = _sc_agg(h, src_p, dst_p, zeros)
        layer = _layer_last if l == NUM_LAYERS - 1 else _layer_mid
        h = layer(h, agg, W1[l], b1[l].reshape(1, D), W2[l], b2[l].reshape(1, D))
    return h


def kernel(x, edge_index, W_in, b_in, W1, b1, W2, b2):
    src = edge_index[0].astype(jnp.int32)
    dst = edge_index[1].astype(jnp.int32)
    pad = E_PAD - E
    pad_dst = N + (jnp.arange(pad, dtype=jnp.int32) % (AGG_ROWS - N))
    src_p = jnp.concatenate([src, jnp.zeros((pad,), jnp.int32)]).reshape(NW, CPT, CH)
    dst_p = jnp.concatenate([dst, pad_dst]).reshape(NW, CPT, CH)
    return _run(x, src_p, dst_p, W_in, b_in, W1, b1, W2, b2)


# speedup vs baseline: 2.8309x; 2.8309x over previous
import functools

import jax
import jax.numpy as jnp
from jax import lax
from jax.experimental import pallas as pl
from jax.experimental.pallas import tpu as pltpu
from jax.experimental.pallas import tpu_sc as plsc

N = 10000
E = 320000
D = 128
NUM_LAYERS = 5

NC = 2
NS = 16
NW = NC * NS
CH = 128
CPT = 80
EPT = CPT * CH
E_PAD = NW * EPT
K = 2
SB = 16
NSB = CPT // SB
ROWS_PER_TILE = 640
AGG_ROWS = NS * ROWS_PER_TILE


DO_GATHER = True
DO_SCATTER = True


def _sc_agg_body(h_hbm, src_hbm, dst_hbm, zeros_hbm, out_hbm,
                 agg_sh, src_v, dst_v, rows_v, sem):
    c = lax.axis_index("c")
    s = lax.axis_index("s")
    wid = c * NS + s

    pltpu.sync_copy(zeros_hbm, agg_sh.at[pl.ds(s * ROWS_PER_TILE, ROWS_PER_TILE)])

    pltpu.sync_copy(src_hbm.at[wid], src_v)
    pltpu.sync_copy(dst_hbm.at[wid], dst_v)
    plsc.subcore_barrier()

    @pl.loop(0, CPT)
    def _(ci):
        if DO_GATHER:
            pltpu.async_copy(h_hbm.at[src_v.at[ci]], rows_v, sem).wait()
        if DO_SCATTER:
            pltpu.sync_copy(rows_v, agg_sh.at[dst_v.at[ci]], add=True)

    plsc.subcore_barrier()
    pltpu.sync_copy(agg_sh.at[pl.ds(s * ROWS_PER_TILE, ROWS_PER_TILE)],
                    out_hbm.at[c, pl.ds(s * ROWS_PER_TILE, ROWS_PER_TILE)])


_sc_agg = pl.kernel(
    _sc_agg_body,
    out_type=jax.ShapeDtypeStruct((NC, AGG_ROWS, D), jnp.float32),
    mesh=plsc.VectorSubcoreMesh(core_axis_name="c", subcore_axis_name="s",
                                num_cores=NC, num_subcores=NS),
    scratch_types=[
        pltpu.VMEM_SHARED((AGG_ROWS, D), jnp.float32),
        pltpu.VMEM((CPT, CH), jnp.int32),
        pltpu.VMEM((CPT, CH), jnp.int32),
        pltpu.VMEM((CH, D), jnp.float32),
        pltpu.SemaphoreType.DMA,
    ],
)


def _encoder_body(x_ref, w_ref, b_ref, o_ref):
    o_ref[...] = (
        jnp.dot(x_ref[...], w_ref[...], preferred_element_type=jnp.float32)
        + b_ref[...]
    )


_encoder = pl.pallas_call(
    _encoder_body,
    out_shape=jax.ShapeDtypeStruct((N, D), jnp.float32),
)


def _layer_body(h_ref, a_ref, w1_ref, b1_ref, w2_ref, b2_ref, o_ref, *, last):
    z = h_ref[...] + a_ref[0, :N, :] + a_ref[1, :N, :]
    z = jnp.dot(z, w1_ref[...], preferred_element_type=jnp.float32) + b1_ref[...]
    z = jnp.maximum(z, 0.0)
    z = jnp.dot(z, w2_ref[...], preferred_element_type=jnp.float32) + b2_ref[...]
    if not last:
        z = jnp.maximum(z, 0.0)
    o_ref[...] = z


def _layer_call(last):
    return pl.pallas_call(
        functools.partial(_layer_body, last=last),
        out_shape=jax.ShapeDtypeStruct((N, D), jnp.float32),
    )


_layer_mid = _layer_call(False)
_layer_last = _layer_call(True)


@jax.jit
def _run(x, src_p, dst_p, W_in, b_in, W1, b1, W2, b2):
    zeros = jnp.zeros((ROWS_PER_TILE, D), jnp.float32)
    h = _encoder(x, W_in, b_in.reshape(1, D))
    for l in range(NUM_LAYERS):
        agg = _sc_agg(h, src_p, dst_p, zeros)
        layer = _layer_last if l == NUM_LAYERS - 1 else _layer_mid
        h = layer(h, agg, W1[l], b1[l].reshape(1, D), W2[l], b2[l].reshape(1, D))
    return h


def kernel(x, edge_index, W_in, b_in, W1, b1, W2, b2):
    src = edge_index[0].astype(jnp.int32)
    dst = edge_index[1].astype(jnp.int32)
    pad = E_PAD - E
    pad_dst = N + (jnp.arange(pad, dtype=jnp.int32) % (AGG_ROWS - N))
    pad_src = jnp.arange(pad, dtype=jnp.int32) % N
    src_p = jnp.concatenate([src, pad_src]).reshape(NW, CPT, CH)
    dst_p = jnp.concatenate([dst, pad_dst]).reshape(NW, CPT, CH)
    return _run(x, src_p, dst_p, W_in, b_in, W1, b1, W2, b2)


# speedup vs baseline: 4.3477x; 1.5358x over previous
import functools

import jax
import jax.numpy as jnp
from jax import lax
from jax.experimental import pallas as pl
from jax.experimental.pallas import tpu as pltpu
from jax.experimental.pallas import tpu_sc as plsc

N = 10000
E = 320000
D = 128
NUM_LAYERS = 5

NC = 2
NS = 16
NW = NC * NS
CH = 128
CPT = 80
EPT = CPT * CH
E_PAD = NW * EPT
K = 2
SB = 16
NSB = CPT // SB
ROWS_PER_TILE = 640
AGG_ROWS = NS * ROWS_PER_TILE


def _sc_agg_body(h_hbm, src_hbm, dst_hbm, zeros_hbm, out_hbm,
                 agg_sh, si0, si1, di0, di1, r0, r1, is0, is1, gs0, gs1):
    si = (si0, si1)
    di = (di0, di1)
    rows = (r0, r1)
    isem = (is0, is1)
    gsem = (gs0, gs1)
    c = lax.axis_index("c")
    s = lax.axis_index("s")
    wid = c * NS + s

    pltpu.sync_copy(zeros_hbm, agg_sh.at[pl.ds(s * ROWS_PER_TILE, ROWS_PER_TILE)])

    def idx_start(sb, b):
        pltpu.async_copy(src_hbm.at[wid, pl.ds(sb * SB, SB)], si[b], isem[b])
        pltpu.async_copy(dst_hbm.at[wid, pl.ds(sb * SB, SB)], di[b], isem[b])

    def idx_wait(sb, b):
        pltpu.make_async_copy(src_hbm.at[wid, pl.ds(sb * SB, SB)], si[b], isem[b]).wait()
        pltpu.make_async_copy(dst_hbm.at[wid, pl.ds(sb * SB, SB)], di[b], isem[b]).wait()

    def gather_start(b, off, k):
        pltpu.async_copy(h_hbm.at[si[b].at[off]], rows[k], gsem[k])

    def gather_wait(b, off, k):
        pltpu.make_async_copy(h_hbm.at[si[b].at[off]], rows[k], gsem[k]).wait()

    def scatter(b, off, k):
        pltpu.sync_copy(rows[k], agg_sh.at[di[b].at[off]], add=True)

    idx_start(0, 0)
    idx_wait(0, 0)
    if NSB > 1:
        idx_start(1, 1)
    plsc.subcore_barrier()

    gather_start(0, 0, 0)
    gather_start(0, 1, 1)

    for sb in range(NSB):
        b = sb % 2
        nb = (sb + 1) % 2

        @pl.loop(0, (SB - K) // K)
        def _(g, b=b):
            base = g * K
            for k in range(K):
                gather_wait(b, base + k, k)
                scatter(b, base + k, k)
                gather_start(b, base + K + k, k)

        if sb + 1 < NSB:
            idx_wait(sb + 1, nb)
            for k in range(K):
                gather_wait(b, SB - K + k, k)
                scatter(b, SB - K + k, k)
                gather_start(nb, k, k)
            if sb + 2 < NSB:
                idx_start(sb + 2, b)
        else:
            for k in range(K):
                gather_wait(b, SB - K + k, k)
                scatter(b, SB - K + k, k)

    plsc.subcore_barrier()
    pltpu.sync_copy(agg_sh.at[pl.ds(s * ROWS_PER_TILE, ROWS_PER_TILE)],
                    out_hbm.at[c, pl.ds(s * ROWS_PER_TILE, ROWS_PER_TILE)])


_sc_agg = pl.kernel(
    _sc_agg_body,
    out_type=jax.ShapeDtypeStruct((NC, AGG_ROWS, D), jnp.float32),
    mesh=plsc.VectorSubcoreMesh(core_axis_name="c", subcore_axis_name="s",
                                num_cores=NC, num_subcores=NS),
    scratch_types=[
        pltpu.VMEM_SHARED((AGG_ROWS, D), jnp.float32),
    ] + [pltpu.VMEM((SB, CH), jnp.int32)] * 4
      + [pltpu.VMEM((CH, D), jnp.float32)] * K
      + [pltpu.SemaphoreType.DMA] * 2
      + [pltpu.SemaphoreType.DMA] * K,
)


def _encoder_body(x_ref, w_ref, b_ref, o_ref):
    o_ref[...] = (
        jnp.dot(x_ref[...], w_ref[...], preferred_element_type=jnp.float32)
        + b_ref[...]
    )


_encoder = pl.pallas_call(
    _encoder_body,
    out_shape=jax.ShapeDtypeStruct((N, D), jnp.float32),
)


def _layer_body(h_ref, a_ref, w1_ref, b1_ref, w2_ref, b2_ref, o_ref, *, last):
    z = h_ref[...] + a_ref[0, :N, :] + a_ref[1, :N, :]
    z = jnp.dot(z, w1_ref[...], preferred_element_type=jnp.float32) + b1_ref[...]
    z = jnp.maximum(z, 0.0)
    z = jnp.dot(z, w2_ref[...], preferred_element_type=jnp.float32) + b2_ref[...]
    if not last:
        z = jnp.maximum(z, 0.0)
    o_ref[...] = z


def _layer_call(last):
    return pl.pallas_call(
        functools.partial(_layer_body, last=last),
        out_shape=jax.ShapeDtypeStruct((N, D), jnp.float32),
    )


_layer_mid = _layer_call(False)
_layer_last = _layer_call(True)


@jax.jit
def _run(x, src_p, dst_p, W_in, b_in, W1, b1, W2, b2):
    zeros = jnp.zeros((ROWS_PER_TILE, D), jnp.float32)
    h = _encoder(x, W_in, b_in.reshape(1, D))
    for l in range(NUM_LAYERS):
        agg = _sc_agg(h, src_p, dst_p, zeros)
        layer = _layer_last if l == NUM_LAYERS - 1 else _layer_mid
        h = layer(h, agg, W1[l], b1[l].reshape(1, D), W2[l], b2[l].reshape(1, D))
    return h


def kernel(x, edge_index, W_in, b_in, W1, b1, W2, b2):
    src = edge_index[0].astype(jnp.int32)
    dst = edge_index[1].astype(jnp.int32)
    pad = E_PAD - E
    pad_dst = N + (jnp.arange(pad, dtype=jnp.int32) % (AGG_ROWS - N))
    pad_src = jnp.arange(pad, dtype=jnp.int32) % N
    src_p = jnp.concatenate([src, pad_src]).reshape(NW, CPT, CH)
    dst_p = jnp.concatenate([dst, pad_dst]).reshape(NW, CPT, CH)
    return _run(x, src_p, dst_p, W_in, b_in, W1, b1, W2, b2)


# speedup vs baseline: 4.7908x; 1.1019x over previous
import functools

import jax
import jax.numpy as jnp
from jax import lax
from jax.experimental import pallas as pl
from jax.experimental.pallas import tpu as pltpu
from jax.experimental.pallas import tpu_sc as plsc

N = 10000
E = 320000
D = 128
NUM_LAYERS = 5

NC = 2
NS = 16
NW = NC * NS
CH = 96
CPT = 105
EPT = CPT * CH
E_PAD = NW * EPT
K = 3
SB = 15
NSB = CPT // SB
ROWS_PER_TILE = 640
AGG_ROWS = NS * ROWS_PER_TILE


def _sc_agg_body(h_hbm, src_hbm, dst_hbm, zeros_hbm, out_hbm,
                 agg_sh, *rest):
    si = rest[0:2]
    di = rest[2:4]
    rows = rest[4:4 + K]
    isem = rest[4 + K:6 + K]
    gsem = rest[6 + K:6 + 2 * K]
    c = lax.axis_index("c")
    s = lax.axis_index("s")
    wid = c * NS + s

    pltpu.sync_copy(zeros_hbm, agg_sh.at[pl.ds(s * ROWS_PER_TILE, ROWS_PER_TILE)])

    def idx_start(sb, b):
        pltpu.async_copy(src_hbm.at[wid, sb], si[b], isem[b])
        pltpu.async_copy(dst_hbm.at[wid, sb], di[b], isem[b])

    def idx_wait(sb, b):
        pltpu.make_async_copy(src_hbm.at[wid, sb], si[b], isem[b]).wait()
        pltpu.make_async_copy(dst_hbm.at[wid, sb], di[b], isem[b]).wait()

    def gather_start(b, off, k):
        pltpu.async_copy(h_hbm.at[si[b].at[off]], rows[k], gsem[k])

    def gather_wait(b, off, k):
        pltpu.make_async_copy(h_hbm.at[si[b].at[off]], rows[k], gsem[k]).wait()

    def scatter(b, off, k):
        pltpu.sync_copy(rows[k], agg_sh.at[di[b].at[off]], add=True)

    idx_start(0, 0)
    idx_wait(0, 0)
    if NSB > 1:
        idx_start(1, 1)
    plsc.subcore_barrier()

    for k in range(K):
        gather_start(0, k, k)

    for sb in range(NSB):
        b = sb % 2
        nb = (sb + 1) % 2

        @pl.loop(0, (SB - K) // K)
        def _(g, b=b):
            base = g * K
            for k in range(K):
                gather_wait(b, base + k, k)
                scatter(b, base + k, k)
                gather_start(b, base + K + k, k)

        if sb + 1 < NSB:
            idx_wait(sb + 1, nb)
            for k in range(K):
                gather_wait(b, SB - K + k, k)
                scatter(b, SB - K + k, k)
                gather_start(nb, k, k)
            if sb + 2 < NSB:
                idx_start(sb + 2, b)
        else:
            for k in range(K):
                gather_wait(b, SB - K + k, k)
                scatter(b, SB - K + k, k)

    plsc.subcore_barrier()
    pltpu.sync_copy(agg_sh.at[pl.ds(s * ROWS_PER_TILE, ROWS_PER_TILE)],
                    out_hbm.at[c, pl.ds(s * ROWS_PER_TILE, ROWS_PER_TILE)])


_sc_agg = pl.kernel(
    _sc_agg_body,
    out_type=jax.ShapeDtypeStruct((NC, AGG_ROWS, D), jnp.float32),
    mesh=plsc.VectorSubcoreMesh(core_axis_name="c", subcore_axis_name="s",
                                num_cores=NC, num_subcores=NS),
    scratch_types=[
        pltpu.VMEM_SHARED((AGG_ROWS, D), jnp.float32),
    ] + [pltpu.VMEM((SB, CH), jnp.int32)] * 4
      + [pltpu.VMEM((CH, D), jnp.float32)] * K
      + [pltpu.SemaphoreType.DMA] * (2 + K),
)


def _encoder_body(x_ref, w_ref, b_ref, o_ref):
    o_ref[...] = (
        jnp.dot(x_ref[...], w_ref[...], preferred_element_type=jnp.float32)
        + b_ref[...]
    )


_encoder = pl.pallas_call(
    _encoder_body,
    out_shape=jax.ShapeDtypeStruct((N, D), jnp.float32),
)


def _layer_body(h_ref, a_ref, w1_ref, b1_ref, w2_ref, b2_ref, o_ref, *, last):
    z = h_ref[...] + a_ref[0, :N, :] + a_ref[1, :N, :]
    z = jnp.dot(z, w1_ref[...], preferred_element_type=jnp.float32) + b1_ref[...]
    z = jnp.maximum(z, 0.0)
    z = jnp.dot(z, w2_ref[...], preferred_element_type=jnp.float32) + b2_ref[...]
    if not last:
        z = jnp.maximum(z, 0.0)
    o_ref[...] = z


def _layer_call(last):
    return pl.pallas_call(
        functools.partial(_layer_body, last=last),
        out_shape=jax.ShapeDtypeStruct((N, D), jnp.float32),
    )


_layer_mid = _layer_call(False)
_layer_last = _layer_call(True)


@jax.jit
def _run(x, src_p, dst_p, W_in, b_in, W1, b1, W2, b2):
    zeros = jnp.zeros((ROWS_PER_TILE, D), jnp.float32)
    h = _encoder(x, W_in, b_in.reshape(1, D))
    for l in range(NUM_LAYERS):
        agg = _sc_agg(h, src_p, dst_p, zeros)
        layer = _layer_last if l == NUM_LAYERS - 1 else _layer_mid
        h = layer(h, agg, W1[l], b1[l].reshape(1, D), W2[l], b2[l].reshape(1, D))
    return h


def kernel(x, edge_index, W_in, b_in, W1, b1, W2, b2):
    src = edge_index[0].astype(jnp.int32)
    dst = edge_index[1].astype(jnp.int32)
    pad = E_PAD - E
    pad_dst = N + (jnp.arange(pad, dtype=jnp.int32) % (AGG_ROWS - N))
    pad_src = jnp.arange(pad, dtype=jnp.int32) % N
    src_p = jnp.concatenate([src, pad_src]).reshape(NW, NSB, SB, CH)
    dst_p = jnp.concatenate([dst, pad_dst]).reshape(NW, NSB, SB, CH)
    return _run(x, src_p, dst_p, W_in, b_in, W1, b1, W2, b2)


# speedup vs baseline: 4.8802x; 1.0187x over previous
import functools

import jax
import jax.numpy as jnp
from jax import lax
from jax.experimental import pallas as pl
from jax.experimental.pallas import tpu as pltpu
from jax.experimental.pallas import tpu_sc as plsc

N = 10000
E = 320000
D = 128
NUM_LAYERS = 5

NC = 2
NS = 16
NW = NC * NS
CH = 80
CPT = 128
EPT = CPT * CH
E_PAD = NW * EPT
K = 4
SB = 16
NSB = CPT // SB
ROWS_PER_TILE = 640
AGG_ROWS = NS * ROWS_PER_TILE


def _sc_agg_body(h_hbm, src_hbm, dst_hbm, zeros_hbm, out_hbm,
                 agg_sh, *rest):
    si = rest[0:2]
    di = rest[2:4]
    rows = rest[4:4 + K]
    isem = rest[4 + K:6 + K]
    gsem = rest[6 + K:6 + 2 * K]
    c = lax.axis_index("c")
    s = lax.axis_index("s")
    wid = c * NS + s

    pltpu.sync_copy(zeros_hbm, agg_sh.at[pl.ds(s * ROWS_PER_TILE, ROWS_PER_TILE)])

    def idx_start(sb, b):
        pltpu.async_copy(src_hbm.at[wid, sb], si[b], isem[b])
        pltpu.async_copy(dst_hbm.at[wid, sb], di[b], isem[b])

    def idx_wait(sb, b):
        pltpu.make_async_copy(src_hbm.at[wid, sb], si[b], isem[b]).wait()
        pltpu.make_async_copy(dst_hbm.at[wid, sb], di[b], isem[b]).wait()

    def gather_start(b, off, k):
        pltpu.async_copy(h_hbm.at[si[b].at[off]], rows[k], gsem[k])

    def gather_wait(b, off, k):
        pltpu.make_async_copy(h_hbm.at[si[b].at[off]], rows[k], gsem[k]).wait()

    def scatter(b, off, k):
        pltpu.sync_copy(rows[k], agg_sh.at[di[b].at[off]], add=True)

    idx_start(0, 0)
    idx_wait(0, 0)
    if NSB > 1:
        idx_start(1, 1)
    plsc.subcore_barrier()

    for k in range(K):
        gather_start(0, k, k)

    for sb in range(NSB):
        b = sb % 2
        nb = (sb + 1) % 2

        @pl.loop(0, (SB - K) // K)
        def _(g, b=b):
            base = g * K
            for k in range(K):
                gather_wait(b, base + k, k)
                scatter(b, base + k, k)
                gather_start(b, base + K + k, k)

        if sb + 1 < NSB:
            idx_wait(sb + 1, nb)
            for k in range(K):
                gather_wait(b, SB - K + k, k)
                scatter(b, SB - K + k, k)
                gather_start(nb, k, k)
            if sb + 2 < NSB:
                idx_start(sb + 2, b)
        else:
            for k in range(K):
                gather_wait(b, SB - K + k, k)
                scatter(b, SB - K + k, k)

    plsc.subcore_barrier()
    pltpu.sync_copy(agg_sh.at[pl.ds(s * ROWS_PER_TILE, ROWS_PER_TILE)],
                    out_hbm.at[c, pl.ds(s * ROWS_PER_TILE, ROWS_PER_TILE)])


_sc_agg = pl.kernel(
    _sc_agg_body,
    out_type=jax.ShapeDtypeStruct((NC, AGG_ROWS, D), jnp.float32),
    mesh=plsc.VectorSubcoreMesh(core_axis_name="c", subcore_axis_name="s",
                                num_cores=NC, num_subcores=NS),
    scratch_types=[
        pltpu.VMEM_SHARED((AGG_ROWS, D), jnp.float32),
    ] + [pltpu.VMEM((SB, CH), jnp.int32)] * 4
      + [pltpu.VMEM((CH, D), jnp.float32)] * K
      + [pltpu.SemaphoreType.DMA] * (2 + K),
)


def _encoder_body(x_ref, w_ref, b_ref, o_ref):
    o_ref[...] = (
        jnp.dot(x_ref[...], w_ref[...], preferred_element_type=jnp.float32)
        + b_ref[...]
    )


_encoder = pl.pallas_call(
    _encoder_body,
    out_shape=jax.ShapeDtypeStruct((N, D), jnp.float32),
)


def _layer_body(h_ref, a_ref, w1_ref, b1_ref, w2_ref, b2_ref, o_ref, *, last):
    z = h_ref[...] + a_ref[0, :N, :] + a_ref[1, :N, :]
    z = jnp.dot(z, w1_ref[...], preferred_element_type=jnp.float32) + b1_ref[...]
    z = jnp.maximum(z, 0.0)
    z = jnp.dot(z, w2_ref[...], preferred_element_type=jnp.float32) + b2_ref[...]
    if not last:
        z = jnp.maximum(z, 0.0)
    o_ref[...] = z


def _layer_call(last):
    return pl.pallas_call(
        functools.partial(_layer_body, last=last),
        out_shape=jax.ShapeDtypeStruct((N, D), jnp.float32),
    )


_layer_mid = _layer_call(False)
_layer_last = _layer_call(True)


@jax.jit
def _run(x, src_p, dst_p, W_in, b_in, W1, b1, W2, b2):
    zeros = jnp.zeros((ROWS_PER_TILE, D), jnp.float32)
    h = _encoder(x, W_in, b_in.reshape(1, D))
    for l in range(NUM_LAYERS):
        agg = _sc_agg(h, src_p, dst_p, zeros)
        layer = _layer_last if l == NUM_LAYERS - 1 else _layer_mid
        h = layer(h, agg, W1[l], b1[l].reshape(1, D), W2[l], b2[l].reshape(1, D))
    return h


def kernel(x, edge_index, W_in, b_in, W1, b1, W2, b2):
    src = edge_index[0].astype(jnp.int32)
    dst = edge_index[1].astype(jnp.int32)
    pad = E_PAD - E
    pad_dst = N + (jnp.arange(pad, dtype=jnp.int32) % (AGG_ROWS - N))
    pad_src = jnp.arange(pad, dtype=jnp.int32) % N
    src_p = jnp.concatenate([src, pad_src]).reshape(NW, NSB, SB, CH)
    dst_p = jnp.concatenate([dst, pad_dst]).reshape(NW, NSB, SB, CH)
    return _run(x, src_p, dst_p, W_in, b_in, W1, b1, W2, b2)


# speedup vs baseline: 4.9014x; 1.0043x over previous
import functools

import jax
import jax.numpy as jnp
from jax import lax
from jax.experimental import pallas as pl
from jax.experimental.pallas import tpu as pltpu
from jax.experimental.pallas import tpu_sc as plsc

N = 10000
E = 320000
D = 128
NUM_LAYERS = 5

NC = 2
NS = 16
NW = NC * NS
CH = 80
CPT = 128
EPT = CPT * CH
E_PAD = NW * EPT
K = 4
SB = 16
NSB = CPT // SB
ROWS_PER_TILE = 640
AGG_ROWS = NS * ROWS_PER_TILE


def _sc_agg_body(h_hbm, src_hbm, dst_hbm, zeros_hbm, out_hbm,
                 agg_sh, *rest):
    si = rest[0:2]
    di = rest[2:4]
    rows = rest[4:4 + K]
    isem = rest[4 + K:6 + K]
    gsem = rest[6 + K:6 + 2 * K]
    c = lax.axis_index("c")
    s = lax.axis_index("s")
    wid = c * NS + s

    pltpu.sync_copy(zeros_hbm, rows[0])
    for j in range(ROWS_PER_TILE // CH):
        pltpu.sync_copy(rows[0], agg_sh.at[pl.ds(s * ROWS_PER_TILE + j * CH, CH)])

    def idx_start(sb, b):
        pltpu.async_copy(src_hbm.at[wid, sb], si[b], isem[b])
        pltpu.async_copy(dst_hbm.at[wid, sb], di[b], isem[b])

    def idx_wait(sb, b):
        pltpu.make_async_copy(src_hbm.at[wid, sb], si[b], isem[b]).wait()
        pltpu.make_async_copy(dst_hbm.at[wid, sb], di[b], isem[b]).wait()

    def gather_start(b, off, k):
        pltpu.async_copy(h_hbm.at[si[b].at[off]], rows[k], gsem[k])

    def gather_wait(b, off, k):
        pltpu.make_async_copy(h_hbm.at[si[b].at[off]], rows[k], gsem[k]).wait()

    def scatter(b, off, k):
        pltpu.sync_copy(rows[k], agg_sh.at[di[b].at[off]], add=True)

    idx_start(0, 0)
    idx_wait(0, 0)
    if NSB > 1:
        idx_start(1, 1)
    plsc.subcore_barrier()

    for k in range(K):
        gather_start(0, k, k)

    for sb in range(NSB):
        b = sb % 2
        nb = (sb + 1) % 2

        @pl.loop(0, (SB - K) // K)
        def _(g, b=b):
            base = g * K
            for k in range(K):
                gather_wait(b, base + k, k)
                scatter(b, base + k, k)
                gather_start(b, base + K + k, k)

        if sb + 1 < NSB:
            idx_wait(sb + 1, nb)
            for k in range(K):
                gather_wait(b, SB - K + k, k)
                scatter(b, SB - K + k, k)
                gather_start(nb, k, k)
            if sb + 2 < NSB:
                idx_start(sb + 2, b)
        else:
            for k in range(K):
                gather_wait(b, SB - K + k, k)
                scatter(b, SB - K + k, k)

    plsc.subcore_barrier()
    pltpu.sync_copy(agg_sh.at[pl.ds(s * ROWS_PER_TILE, ROWS_PER_TILE)],
                    out_hbm.at[c, pl.ds(s * ROWS_PER_TILE, ROWS_PER_TILE)])


_sc_agg = pl.kernel(
    _sc_agg_body,
    out_type=jax.ShapeDtypeStruct((NC, AGG_ROWS, D), jnp.float32),
    mesh=plsc.VectorSubcoreMesh(core_axis_name="c", subcore_axis_name="s",
                                num_cores=NC, num_subcores=NS),
    scratch_types=[
        pltpu.VMEM_SHARED((AGG_ROWS, D), jnp.float32),
    ] + [pltpu.VMEM((SB, CH), jnp.int32)] * 4
      + [pltpu.VMEM((CH, D), jnp.float32)] * K
      + [pltpu.SemaphoreType.DMA] * (2 + K),
)


def _encoder_body(x_ref, w_ref, b_ref, o_ref):
    o_ref[...] = (
        jnp.dot(x_ref[...], w_ref[...], preferred_element_type=jnp.float32)
        + b_ref[...]
    )


_encoder = pl.pallas_call(
    _encoder_body,
    out_shape=jax.ShapeDtypeStruct((N, D), jnp.float32),
)


def _layer_body(h_ref, a_ref, w1_ref, b1_ref, w2_ref, b2_ref, o_ref, *, last):
    z = h_ref[...] + a_ref[0] + a_ref[1]
    z = jnp.dot(z, w1_ref[...], preferred_element_type=jnp.float32) + b1_ref[...]
    z = jnp.maximum(z, 0.0)
    z = jnp.dot(z, w2_ref[...], preferred_element_type=jnp.float32) + b2_ref[...]
    if not last:
        z = jnp.maximum(z, 0.0)
    o_ref[...] = z


RB = 2000


def _layer_call(last):
    return pl.pallas_call(
        functools.partial(_layer_body, last=last),
        grid=(N // RB,),
        in_specs=[
            pl.BlockSpec((RB, D), lambda i: (i, 0)),
            pl.BlockSpec((NC, RB, D), lambda i: (0, i, 0)),
            pl.BlockSpec((D, D), lambda i: (0, 0)),
            pl.BlockSpec((1, D), lambda i: (0, 0)),
            pl.BlockSpec((D, D), lambda i: (0, 0)),
            pl.BlockSpec((1, D), lambda i: (0, 0)),
        ],
        out_specs=pl.BlockSpec((RB, D), lambda i: (i, 0)),
        out_shape=jax.ShapeDtypeStruct((N, D), jnp.float32),
    )


_layer_mid = _layer_call(False)
_layer_last = _layer_call(True)


@jax.jit
def _run(x, src_p, dst_p, W_in, b_in, W1, b1, W2, b2):
    zeros = jnp.zeros((CH, D), jnp.float32)
    h = _encoder(x, W_in, b_in.reshape(1, D))
    for l in range(NUM_LAYERS):
        agg = _sc_agg(h, src_p, dst_p, zeros)
        layer = _layer_last if l == NUM_LAYERS - 1 else _layer_mid
        h = layer(h, agg, W1[l], b1[l].reshape(1, D), W2[l], b2[l].reshape(1, D))
    return h


def kernel(x, edge_index, W_in, b_in, W1, b1, W2, b2):
    src = edge_index[0].astype(jnp.int32)
    dst = edge_index[1].astype(jnp.int32)
    pad = E_PAD - E
    pad_dst = N + (jnp.arange(pad, dtype=jnp.int32) % (AGG_ROWS - N))
    pad_src = jnp.arange(pad, dtype=jnp.int32) % N
    src_p = jnp.concatenate([src, pad_src]).reshape(NW, NSB, SB, CH)
    dst_p = jnp.concatenate([dst, pad_dst]).reshape(NW, NSB, SB, CH)
    return _run(x, src_p, dst_p, W_in, b_in, W1, b1, W2, b2)
